# Initial kernel scaffold; baseline (speedup 1.0000x reference)
import jax
import jax.numpy as jnp
from jax.experimental import pallas as pl


def kernel(token_type_ids, token_type_embeddings):
    raise NotImplementedError("write your pallas kernel here")


# speedup vs baseline: 1.0326x; 1.0326x over previous
import jax
import jax.numpy as jnp
from jax.experimental import pallas as pl
from jax.experimental.pallas import tpu as pltpu

_H = 4096
_N = 4 * 4096
_R = 512


def _tc_body(ids_ref, tab_ref, out_ref):
    i = pl.program_id(0)
    ids = ids_ref[0, pl.ds(i * _R, _R)]
    f = ids.astype(jnp.float32)[:, None]
    row0 = tab_ref[0:1, :]
    diff = tab_ref[1:2, :] - tab_ref[0:1, :]
    out_ref[...] = row0 + f * diff


def kernel(token_type_ids, token_type_embeddings):
    flat = token_type_ids.reshape(1, _N)
    out = pl.pallas_call(
        _tc_body,
        grid=(_N // _R,),
        in_specs=[
            pl.BlockSpec((1, _N), lambda i: (0, 0)),
            pl.BlockSpec((2, _H), lambda i: (0, 0)),
        ],
        out_specs=pl.BlockSpec((_R, _H), lambda i: (i, 0)),
        out_shape=jax.ShapeDtypeStruct((_N, _H), jnp.float32),
    )(flat, token_type_embeddings)
    return out.reshape(token_type_ids.shape + (_H,))
